# baseline (speedup 1.0000x reference)
import functools
import jax
import jax.numpy as jnp
from jax import lax
from jax.experimental import pallas as pl
from jax.experimental.pallas import tpu as pltpu
from jax.experimental.pallas import tpu_sc as plsc

_B = 16384
_F = 16
_NW = 32
_BPW = _B // _NW
_W = 2048
_NFULL = 488
_RAG0 = _NFULL * _W
_RAGW = 512
_TAIL0 = _RAG0 + _RAGW
_NWIN = _NFULL + 1
_SENT = 1 << 30
_IMG = (_B + 16) * _F


def _scan_body(uidx_hbm, iidx_hbm, ut_hbm, it_hbm,
               ue_hbm, ve_hbm,
               uidx_v, iidx_v, win_v, cu_i, cu_b, ci_i, ci_b,
               hw_i, hw_b, stage_v, sidx_v,
               sem, sem_i):
    wid = lax.axis_index("s") * 2 + lax.axis_index("c")
    wlo = (_NWIN * wid) // _NW
    whi = (_NWIN * (wid + 1)) // _NW
    whi_full = jnp.minimum(whi, _NFULL)
    lane = lax.iota(jnp.int32, 16)

    pltpu.sync_copy(uidx_hbm, uidx_v)
    pltpu.sync_copy(iidx_hbm, iidx_v)

    def cand(k, carry):
        nu, ni = carry
        sl = pl.ds(k * 16, 16)
        bvec = lane + k * 16
        uv = uidx_v[sl]
        wv = lax.shift_right_logical(uv, 11)
        mu = (wv >= wlo) & (wv < whi) & (uv < _TAIL0)
        plsc.store_compressed(cu_i.at[pl.ds(nu, 16)], uv, mask=mu)
        plsc.store_compressed(cu_b.at[pl.ds(nu, 16)], bvec, mask=mu)
        nu = nu + plsc.all_reduce_population_count(mu)[0]
        iv = iidx_v[sl]
        wv = lax.shift_right_logical(iv, 11)
        mi = (wv >= wlo) & (wv < whi) & (iv < _TAIL0)
        plsc.store_compressed(ci_i.at[pl.ds(ni, 16)], iv, mask=mi)
        plsc.store_compressed(ci_b.at[pl.ds(ni, 16)], bvec, mask=mi)
        ni = ni + plsc.all_reduce_population_count(mi)[0]
        return nu, ni

    nu, ni = lax.fori_loop(0, _B // 16, cand, (jnp.int32(0), jnp.int32(0)))
    sent = jnp.full((16,), _SENT, jnp.int32)
    zero16 = jnp.zeros((16,), jnp.int32)
    cu_i[pl.ds(nu, 16)] = sent
    cu_b[pl.ds(nu, 16)] = zero16
    ci_i[pl.ds(ni, 16)] = sent
    ci_b[pl.ds(ni, 16)] = zero16

    def fire_u(w, buf):
        off = pl.multiple_of(w * _W, 128)
        pltpu.async_copy(ut_hbm.at[:, pl.ds(off, _W)], win_v.at[buf], sem)

    def fire_i(w, buf):
        off = pl.multiple_of(w * _W, 128)
        pltpu.async_copy(it_hbm.at[:, pl.ds(off, _W)], win_v.at[buf], sem_i)

    def wait_slab(buf):
        s = sem if buf == 0 else sem_i
        pltpu.make_async_copy(ut_hbm.at[:, pl.ds(0, _W)], win_v.at[buf],
                              s).wait()

    def process(w, wbase, buf, c_i, c_b, nc, out_hbm):
        def scan_c(k, nh):
            sl = pl.ds(k * 16, 16)
            cv = c_i[sl]
            cb = c_b[sl]
            m = lax.shift_right_logical(cv, 11) == w
            plsc.store_compressed(hw_i.at[pl.ds(nh, 16)], cv, mask=m)
            plsc.store_compressed(hw_b.at[pl.ds(nh, 16)], cb, mask=m)
            return nh + plsc.all_reduce_population_count(m)[0]

        nh = lax.fori_loop(0, (nc + 15) // 16, scan_c, jnp.int32(0))
        hw_i[pl.ds(nh, 16)] = jnp.full((16,), wbase, jnp.int32)
        hw_b[pl.ds(nh, 16)] = jnp.full((16,), _B, jnp.int32)

        def hits(h, carry):
            hv = hw_i[pl.ds(h * 16, 16)]
            hb = hw_b[pl.ds(h * 16, 16)]
            col = hv - wbase
            for half in range(2):
                for r in range(8):
                    q = half * 8 + r
                    vec = plsc.load_gather(
                        win_v.at[buf],
                        [lane, jnp.full((16,), col[q], jnp.int32)])
                    stage_v[pl.ds(r * 16, 16)] = vec
                    sidx_v[0, pl.ds(r * 16, 16)] = hb[q] * 16 + lane
                pltpu.sync_copy(stage_v, out_hbm.at[sidx_v.at[0]])
            return carry

        lax.fori_loop(0, (nh + 15) // 16, hits, 0)

    @pl.when(wlo < whi_full)
    def _():
        fire_u(wlo, 0)
        fire_i(wlo, 1)

    def window(k, carry):
        w = wlo + k
        wait_slab(0)
        process(w, w * _W, 0, cu_i, cu_b, nu, ue_hbm)

        @pl.when(k + 1 < whi_full - wlo)
        def _():
            fire_u(w + 1, 0)

        wait_slab(1)
        process(w, w * _W, 1, ci_i, ci_b, ni, ve_hbm)

        @pl.when(k + 1 < whi_full - wlo)
        def _():
            fire_i(w + 1, 1)

        return carry

    lax.fori_loop(0, whi_full - wlo, window, 0)

    @pl.when(whi == _NWIN)
    def _():
        pltpu.sync_copy(ut_hbm.at[:, pl.ds(_RAG0, _RAGW)],
                        win_v.at[0, :, pl.ds(0, _RAGW)])
        process(jnp.int32(_NFULL), _RAG0, 0, cu_i, cu_b, nu, ue_hbm)
        pltpu.sync_copy(it_hbm.at[:, pl.ds(_RAG0, _RAGW)],
                        win_v.at[1, :, pl.ds(0, _RAGW)])
        process(jnp.int32(_NFULL), _RAG0, 1, ci_i, ci_b, ni, ve_hbm)


def _dot_body(uidx_hbm, iidx_hbm, ue_hbm, ve_hbm, utail_hbm, itail_hbm,
              ubt_hbm, ibt_hbm, gb_hbm,
              out_hbm,
              uidx_v, iidx_v, ue_l, ve_l, ub_v, ib_v, out_v, gb_v,
              utail_v, itail_v, sem):
    wid = lax.axis_index("s") * 2 + lax.axis_index("c")
    base = wid * _BPW
    lane = lax.iota(jnp.int32, 16)

    pltpu.sync_copy(uidx_hbm.at[pl.ds(base, _BPW)], uidx_v)
    pltpu.sync_copy(iidx_hbm.at[pl.ds(base, _BPW)], iidx_v)
    pltpu.sync_copy(ue_hbm.at[pl.ds(base * 16, _BPW * 16)], ue_l)
    pltpu.sync_copy(ve_hbm.at[pl.ds(base * 16, _BPW * 16)], ve_l)
    pltpu.sync_copy(gb_hbm, gb_v)
    pltpu.sync_copy(utail_hbm, utail_v)
    pltpu.sync_copy(itail_hbm, itail_v)
    copies = []
    for c in range(_BPW // 128):
        sl = pl.ds(c * 128, 128)
        copies.append(
            pltpu.async_copy(ubt_hbm.at[uidx_v.at[sl]], ub_v.at[sl], sem))
        copies.append(
            pltpu.async_copy(ibt_hbm.at[iidx_v.at[sl]], ib_v.at[sl], sem))
    for cp in copies:
        cp.wait()
    gb = gb_v[...]

    def block(j, carry):
        r0 = j * 16
        uiv = uidx_v[pl.ds(r0, 16)]
        iiv = iidx_v[pl.ds(r0, 16)]
        acc = gb
        for r in range(16):
            u = ue_l[pl.ds((r0 + r) * 16, 16)]
            v = ve_l[pl.ds((r0 + r) * 16, 16)]
            ut_ix = jnp.maximum(uiv[r] - _TAIL0, 0) * 16 + lane
            it_ix = jnp.maximum(iiv[r] - _TAIL0, 0) * 16 + lane
            u = jnp.where(uiv[r] >= _TAIL0,
                          plsc.load_gather(utail_v, [ut_ix]), u)
            v = jnp.where(iiv[r] >= _TAIL0,
                          plsc.load_gather(itail_v, [it_ix]), v)
            s = jnp.sum(u * v)
            acc = jnp.where(lane == r, acc + s, acc)
        sl = pl.ds(r0, 16)
        out_v[sl] = acc + ub_v[sl] + ib_v[sl]
        return carry

    lax.fori_loop(0, _BPW // 16, block, 0)
    pltpu.sync_copy(out_v, out_hbm.at[pl.ds(base, _BPW)])


def kernel(user_idx, item_idx, user_table, item_table, implicit_table,
           user_bias_table, item_bias_table, global_bias):
    del implicit_table
    uidx = user_idx.astype(jnp.int32)
    iidx = item_idx.astype(jnp.int32)
    ut = user_table.T
    it = item_table.T
    utail = user_table[_TAIL0:].reshape(-1)
    itail = item_table[_TAIL0:].reshape(-1)
    ubt = user_bias_table.reshape(-1)
    ibt = item_bias_table.reshape(-1)
    gb16 = jnp.broadcast_to(global_bias, (16,)).astype(jnp.float32)

    mesh = plsc.VectorSubcoreMesh(core_axis_name="c", subcore_axis_name="s")
    params = pltpu.CompilerParams(needs_layout_passes=False)
    scan = pl.kernel(
        _scan_body,
        out_type=(jax.ShapeDtypeStruct((_IMG,), jnp.float32),
                  jax.ShapeDtypeStruct((_IMG,), jnp.float32)),
        mesh=mesh,
        compiler_params=params,
        scratch_types=[
            pltpu.VMEM((_B,), jnp.int32),
            pltpu.VMEM((_B,), jnp.int32),
            pltpu.VMEM((2, _F, _W), jnp.float32),
            pltpu.VMEM((768,), jnp.int32),
            pltpu.VMEM((768,), jnp.int32),
            pltpu.VMEM((768,), jnp.int32),
            pltpu.VMEM((768,), jnp.int32),
            pltpu.VMEM((768,), jnp.int32),
            pltpu.VMEM((768,), jnp.int32),
            pltpu.VMEM((128,), jnp.float32),
            pltpu.VMEM((1, 128), jnp.int32),
            pltpu.SemaphoreType.DMA,
            pltpu.SemaphoreType.DMA,
        ],
    )
    ue, ve = scan(uidx, iidx, ut, it)
    dot = pl.kernel(
        _dot_body,
        out_type=jax.ShapeDtypeStruct((_B,), jnp.float32),
        mesh=mesh,
        compiler_params=params,
        scratch_types=[
            pltpu.VMEM((_BPW,), jnp.int32),
            pltpu.VMEM((_BPW,), jnp.int32),
            pltpu.VMEM((_BPW * _F,), jnp.float32),
            pltpu.VMEM((_BPW * _F,), jnp.float32),
            pltpu.VMEM((_BPW,), jnp.float32),
            pltpu.VMEM((_BPW,), jnp.float32),
            pltpu.VMEM((_BPW,), jnp.float32),
            pltpu.VMEM((16,), jnp.float32),
            pltpu.VMEM((64 * _F,), jnp.float32),
            pltpu.VMEM((64 * _F,), jnp.float32),
            pltpu.SemaphoreType.DMA,
        ],
    )
    return dot(uidx, iidx, ue, ve, utail, itail, ubt, ibt, gb16)


# speedup vs baseline: 61.2910x; 61.2910x over previous
import functools
import jax
import jax.numpy as jnp
from jax import lax
from jax.experimental import pallas as pl
from jax.experimental.pallas import tpu as pltpu
from jax.experimental.pallas import tpu_sc as plsc

_B = 16384
_F = 16
_NW = 32
_BPW = _B // _NW
_W = 1024
_NFULL = 976
_RAG0 = _NFULL * _W
_RAGW = 512
_TAIL0 = _RAG0 + _RAGW
_NWIN = _NFULL + 1
_HALF = 488
_SPLIT = _HALF * _W
_SENT = 1 << 30
_IMG = (_B + 16) * _F
_G = 8
_CG = 768


def _scan_body(uidx_hbm, iidx_hbm, ut_hbm, it_hbm,
               ue_hbm, ve_hbm,
               uidx_v, iidx_v, win_v, cu_i, cu_b, ci_i, ci_b,
               gu_i, gu_b, gi_i, gi_b,
               hw_i, hw_b, stage_v, sidx_v, bounce_v,
               shu, shi,
               sem, sem_i):
    t = lax.axis_index("s")
    cid = lax.axis_index("c")
    nwin_sc = jnp.where(cid == 0, _HALF, _NWIN - _HALF)
    sc0 = cid * _HALF
    wlo = sc0 + (nwin_sc * t) // 16
    whi = sc0 + (nwin_sc * (t + 1)) // 16
    whi_full = jnp.minimum(whi, _NFULL)
    nw = whi - wlo
    lane = lax.iota(jnp.int32, 16)

    def cand_chunk(ch, carry):
        pltpu.sync_copy(uidx_hbm.at[pl.ds(ch * 2048, 2048)], uidx_v)
        pltpu.sync_copy(iidx_hbm.at[pl.ds(ch * 2048, 2048)], iidx_v)

        def cand(k, carry2):
            nu, ni = carry2
            sl = pl.ds(k * 16, 16)
            bvec = lane + (ch * 2048 + k * 16)
            uv = uidx_v[sl]
            wv = lax.shift_right_logical(uv, 10)
            mu = (wv >= wlo) & (wv < whi) & (uv < _TAIL0)
            plsc.store_compressed(cu_i.at[pl.ds(nu, 16)], uv, mask=mu)
            plsc.store_compressed(cu_b.at[pl.ds(nu, 16)], bvec, mask=mu)
            nu = nu + plsc.all_reduce_population_count(mu)[0]
            iv = iidx_v[sl]
            wv = lax.shift_right_logical(iv, 10)
            mi = (wv >= wlo) & (wv < whi) & (iv < _TAIL0)
            plsc.store_compressed(ci_i.at[pl.ds(ni, 16)], iv, mask=mi)
            plsc.store_compressed(ci_b.at[pl.ds(ni, 16)], bvec, mask=mi)
            ni = ni + plsc.all_reduce_population_count(mi)[0]
            return nu, ni

        return lax.fori_loop(0, 2048 // 16, cand, carry)

    nu, ni = lax.fori_loop(0, _B // 2048, cand_chunk,
                           (jnp.int32(0), jnp.int32(0)))
    sent = jnp.full((16,), _SENT, jnp.int32)
    zero16 = jnp.zeros((16,), jnp.int32)
    cu_i[pl.ds(nu, 16)] = sent
    cu_b[pl.ds(nu, 16)] = zero16
    ci_i[pl.ds(ni, 16)] = sent
    ci_b[pl.ds(ni, 16)] = zero16

    def part(c_i, c_b, nc, g_i, g_b):
        counts = []
        for g in range(_G):
            glo = wlo + (nw * g) // _G
            ghi = wlo + (nw * (g + 1)) // _G

            def one(k, ng):
                sl = pl.ds(k * 16, 16)
                cv = c_i[sl]
                cb = c_b[sl]
                wv = lax.shift_right_logical(cv, 10)
                m = (wv >= glo) & (wv < ghi)
                plsc.store_compressed(g_i.at[pl.ds(g * _CG + ng, 16)], cv,
                                      mask=m)
                plsc.store_compressed(g_b.at[pl.ds(g * _CG + ng, 16)], cb,
                                      mask=m)
                return ng + plsc.all_reduce_population_count(m)[0]

            ng = lax.fori_loop(0, (nc + 15) // 16, one, jnp.int32(0))
            g_i[pl.ds(g * _CG + ng, 16)] = sent
            g_b[pl.ds(g * _CG + ng, 16)] = zero16
            counts.append(ng)
        return counts

    ucnt = part(cu_i, cu_b, nu, gu_i, gu_b)
    icnt = part(ci_i, ci_b, ni, gi_i, gi_b)

    glos = [wlo + (nw * g) // _G for g in range(1, _G)]

    def group_of(w):
        g = jnp.int32(0)
        for b in glos:
            g = g + jnp.where(w >= b, 1, 0).astype(jnp.int32)
        return g

    def count_for(counts, g):
        c = counts[0]
        for gg in range(1, _G):
            c = jnp.where(g == gg, counts[gg], c)
        return c

    def fire_u(w, buf):
        off = pl.multiple_of(w * _W, 128)
        pltpu.async_copy(ut_hbm.at[:, pl.ds(off, _W)], win_v.at[buf], sem)

    def fire_i(w, buf):
        off = pl.multiple_of(w * _W, 128)
        pltpu.async_copy(it_hbm.at[:, pl.ds(off, _W)], win_v.at[buf], sem_i)

    def wait_slab(buf):
        s = sem if buf == 0 else sem_i
        pltpu.make_async_copy(ut_hbm.at[:, pl.ds(0, _W)], win_v.at[buf],
                              s).wait()

    def process(w, wbase, buf, g_i, g_b, counts, sh):
        g = group_of(w)
        g0 = g * _CG
        nc = count_for(counts, g)

        def scan_c(k, nh):
            sl = pl.ds(g0 + k * 16, 16)
            cv = g_i[sl]
            cb = g_b[sl]
            m = lax.shift_right_logical(cv, 10) == w
            plsc.store_compressed(hw_i.at[pl.ds(nh, 16)], cv, mask=m)
            plsc.store_compressed(hw_b.at[pl.ds(nh, 16)], cb, mask=m)
            return nh + plsc.all_reduce_population_count(m)[0]

        nh = lax.fori_loop(0, (nc + 15) // 16, scan_c, jnp.int32(0))
        hw_i[pl.ds(nh, 16)] = jnp.full((16,), wbase, jnp.int32)
        hw_b[pl.ds(nh, 16)] = jnp.full((16,), _B, jnp.int32)

        def hits(h, carry):
            hv = hw_i[pl.ds(h * 16, 16)]
            hb = hw_b[pl.ds(h * 16, 16)]
            col = hv - wbase
            for half in range(2):
                for r in range(8):
                    q = half * 8 + r
                    vec = plsc.load_gather(
                        win_v.at[buf],
                        [lane, jnp.full((16,), col[q], jnp.int32)])
                    stage_v[pl.ds(r * 16, 16)] = vec
                    sidx_v[pl.ds(r * 16, 16)] = hb[q] * 16 + lane
                pltpu.sync_copy(stage_v, sh.at[sidx_v])
            return carry

        lax.fori_loop(0, (nh + 15) // 16, hits, 0)

    @pl.when(wlo < whi_full)
    def _():
        fire_u(wlo, 0)
        fire_i(wlo, 1)

    def window(k, carry):
        w = wlo + k
        wait_slab(0)
        process(w, w * _W, 0, gu_i, gu_b, ucnt, shu)

        @pl.when(k + 1 < whi_full - wlo)
        def _():
            fire_u(w + 1, 0)

        wait_slab(1)
        process(w, w * _W, 1, gi_i, gi_b, icnt, shi)

        @pl.when(k + 1 < whi_full - wlo)
        def _():
            fire_i(w + 1, 1)

        return carry

    lax.fori_loop(0, whi_full - wlo, window, 0)

    @pl.when(whi == _NWIN)
    def _():
        pltpu.sync_copy(ut_hbm.at[:, pl.ds(_RAG0, _RAGW)],
                        win_v.at[0, :, pl.ds(0, _RAGW)])
        process(jnp.int32(_NFULL), _RAG0, 0, gu_i, gu_b, ucnt, shu)
        pltpu.sync_copy(it_hbm.at[:, pl.ds(_RAG0, _RAGW)],
                        win_v.at[1, :, pl.ds(0, _RAGW)])
        process(jnp.int32(_NFULL), _RAG0, 1, gi_i, gi_b, icnt, shi)

    plsc.subcore_barrier()
    seg = _IMG // 16
    for sh, out in ((shu, ue_hbm), (shi, ve_hbm)):
        for c in range(5):
            off = c * 4096
            sz = 4096 if c < 4 else 16
            bsl = pl.ds(0, sz)
            pltpu.sync_copy(sh.at[pl.ds(t * seg + off, sz)], bounce_v.at[bsl])
            pltpu.sync_copy(bounce_v.at[bsl],
                            out.at[pl.ds(cid * _IMG + t * seg + off, sz)])


def _dot_body(uidx_hbm, iidx_hbm, ue_hbm, ve_hbm, utail_hbm, itail_hbm,
              ubt_hbm, ibt_hbm, gb_hbm,
              out_hbm,
              uidx_v, iidx_v, ue0_l, ue1_l, ve0_l, ve1_l,
              ub_v, ib_v, out_v, gb_v, utail_v, itail_v, sem):
    wid = lax.axis_index("s") * 2 + lax.axis_index("c")
    base = wid * _BPW
    lane = lax.iota(jnp.int32, 16)

    pltpu.sync_copy(uidx_hbm.at[pl.ds(base, _BPW)], uidx_v)
    pltpu.sync_copy(iidx_hbm.at[pl.ds(base, _BPW)], iidx_v)
    esl0 = pl.ds(base * 16, _BPW * 16)
    esl1 = pl.ds(_IMG + base * 16, _BPW * 16)
    pltpu.sync_copy(ue_hbm.at[esl0], ue0_l)
    pltpu.sync_copy(ue_hbm.at[esl1], ue1_l)
    pltpu.sync_copy(ve_hbm.at[esl0], ve0_l)
    pltpu.sync_copy(ve_hbm.at[esl1], ve1_l)
    pltpu.sync_copy(gb_hbm, gb_v)
    pltpu.sync_copy(utail_hbm, utail_v)
    pltpu.sync_copy(itail_hbm, itail_v)
    copies = []
    for c in range(_BPW // 128):
        sl = pl.ds(c * 128, 128)
        copies.append(
            pltpu.async_copy(ubt_hbm.at[uidx_v.at[sl]], ub_v.at[sl], sem))
        copies.append(
            pltpu.async_copy(ibt_hbm.at[iidx_v.at[sl]], ib_v.at[sl], sem))
    for cp in copies:
        cp.wait()
    gb = gb_v[...]

    def block(j, carry):
        r0 = j * 16
        uiv = uidx_v[pl.ds(r0, 16)]
        iiv = iidx_v[pl.ds(r0, 16)]
        acc = gb
        for r in range(16):
            rsl = pl.ds((r0 + r) * 16, 16)
            u = jnp.where(uiv[r] < _SPLIT, ue0_l[rsl], ue1_l[rsl])
            v = jnp.where(iiv[r] < _SPLIT, ve0_l[rsl], ve1_l[rsl])
            ut_ix = jnp.maximum(uiv[r] - _TAIL0, 0) * 16 + lane
            it_ix = jnp.maximum(iiv[r] - _TAIL0, 0) * 16 + lane
            u = jnp.where(uiv[r] >= _TAIL0,
                          plsc.load_gather(utail_v, [ut_ix]), u)
            v = jnp.where(iiv[r] >= _TAIL0,
                          plsc.load_gather(itail_v, [it_ix]), v)
            s = jnp.sum(u * v)
            acc = jnp.where(lane == r, acc + s, acc)
        sl = pl.ds(r0, 16)
        out_v[sl] = acc + ub_v[sl] + ib_v[sl]
        return carry

    lax.fori_loop(0, _BPW // 16, block, 0)
    pltpu.sync_copy(out_v, out_hbm.at[pl.ds(base, _BPW)])


def kernel(user_idx, item_idx, user_table, item_table, implicit_table,
           user_bias_table, item_bias_table, global_bias):
    del implicit_table
    uidx = user_idx.astype(jnp.int32)
    iidx = item_idx.astype(jnp.int32)
    ut = user_table.T
    it = item_table.T
    utail = user_table[_TAIL0:].reshape(-1)
    itail = item_table[_TAIL0:].reshape(-1)
    ubt = user_bias_table.reshape(-1)
    ibt = item_bias_table.reshape(-1)
    gb16 = jnp.broadcast_to(global_bias, (16,)).astype(jnp.float32)

    mesh = plsc.VectorSubcoreMesh(core_axis_name="c", subcore_axis_name="s")
    params = pltpu.CompilerParams(needs_layout_passes=False)
    scan = pl.kernel(
        _scan_body,
        out_type=(jax.ShapeDtypeStruct((2 * _IMG,), jnp.float32),
                  jax.ShapeDtypeStruct((2 * _IMG,), jnp.float32)),
        mesh=mesh,
        compiler_params=params,
        scratch_types=[
            pltpu.VMEM((2048,), jnp.int32),
            pltpu.VMEM((2048,), jnp.int32),
            pltpu.VMEM((2, _F, _W), jnp.float32),
            pltpu.VMEM((1168,), jnp.int32),
            pltpu.VMEM((1168,), jnp.int32),
            pltpu.VMEM((1168,), jnp.int32),
            pltpu.VMEM((1168,), jnp.int32),
            pltpu.VMEM((_G * _CG + 16,), jnp.int32),
            pltpu.VMEM((_G * _CG + 16,), jnp.int32),
            pltpu.VMEM((_G * _CG + 16,), jnp.int32),
            pltpu.VMEM((_G * _CG + 16,), jnp.int32),
            pltpu.VMEM((528,), jnp.int32),
            pltpu.VMEM((528,), jnp.int32),
            pltpu.VMEM((128,), jnp.float32),
            pltpu.VMEM((128,), jnp.int32),
            pltpu.VMEM((4096,), jnp.float32),
            pltpu.VMEM_SHARED((_IMG,), jnp.float32),
            pltpu.VMEM_SHARED((_IMG,), jnp.float32),
            pltpu.SemaphoreType.DMA,
            pltpu.SemaphoreType.DMA,
        ],
    )
    ue, ve = scan(uidx, iidx, ut, it)
    dot = pl.kernel(
        _dot_body,
        out_type=jax.ShapeDtypeStruct((_B,), jnp.float32),
        mesh=mesh,
        compiler_params=params,
        scratch_types=[
            pltpu.VMEM((_BPW,), jnp.int32),
            pltpu.VMEM((_BPW,), jnp.int32),
            pltpu.VMEM((_BPW * _F,), jnp.float32),
            pltpu.VMEM((_BPW * _F,), jnp.float32),
            pltpu.VMEM((_BPW * _F,), jnp.float32),
            pltpu.VMEM((_BPW * _F,), jnp.float32),
            pltpu.VMEM((_BPW,), jnp.float32),
            pltpu.VMEM((_BPW,), jnp.float32),
            pltpu.VMEM((_BPW,), jnp.float32),
            pltpu.VMEM((16,), jnp.float32),
            pltpu.VMEM((64 * _F,), jnp.float32),
            pltpu.VMEM((64 * _F,), jnp.float32),
            pltpu.SemaphoreType.DMA,
        ],
    )
    return dot(uidx, iidx, ue, ve, utail, itail, ubt, ibt, gb16)


# speedup vs baseline: 62.8291x; 1.0251x over previous
import functools
import jax
import jax.numpy as jnp
from jax import lax
from jax.experimental import pallas as pl
from jax.experimental.pallas import tpu as pltpu
from jax.experimental.pallas import tpu_sc as plsc

_B = 16384
_F = 16
_NW = 32
_BPW = _B // _NW
_W = 1024
_NFULL = 976
_RAG0 = _NFULL * _W
_RAGW = 512
_TAIL0 = _RAG0 + _RAGW
_NWIN = _NFULL + 1
_HALF = 488
_SPLIT = _HALF * _W
_SENT = 1 << 30
_IMG = (_B + 16) * _F
_G = 8
_CG = 768


def _scan_body(uidx_hbm, iidx_hbm, ut_hbm, it_hbm,
               ue_hbm, ve_hbm,
               uidx_v, iidx_v, win_v, cu_i, cu_b, ci_i, ci_b,
               gu_i, gu_b, gi_i, gi_b,
               hw_i, hw_b, stage_v, sidx_v, bounce_v,
               shu, shi,
               sem, sem_i):
    t = lax.axis_index("s")
    cid = lax.axis_index("c")
    nwin_sc = jnp.where(cid == 0, _HALF, _NWIN - _HALF)
    sc0 = cid * _HALF
    wlo = sc0 + (nwin_sc * t) // 16
    whi = sc0 + (nwin_sc * (t + 1)) // 16
    whi_full = jnp.minimum(whi, _NFULL)
    nw = whi - wlo
    lane = lax.iota(jnp.int32, 16)

    def cand_chunk(ch, carry):
        pltpu.sync_copy(uidx_hbm.at[pl.ds(ch * 2048, 2048)], uidx_v)
        pltpu.sync_copy(iidx_hbm.at[pl.ds(ch * 2048, 2048)], iidx_v)

        def cand(k, carry2):
            nu, ni = carry2
            sl = pl.ds(k * 16, 16)
            bvec = lane + (ch * 2048 + k * 16)
            uv = uidx_v[sl]
            wv = lax.shift_right_logical(uv, 10)
            mu = (wv >= wlo) & (wv < whi) & (uv < _TAIL0)
            plsc.store_compressed(cu_i.at[pl.ds(nu, 16)], uv, mask=mu)
            plsc.store_compressed(cu_b.at[pl.ds(nu, 16)], bvec, mask=mu)
            nu = nu + plsc.all_reduce_population_count(mu)[0]
            iv = iidx_v[sl]
            wv = lax.shift_right_logical(iv, 10)
            mi = (wv >= wlo) & (wv < whi) & (iv < _TAIL0)
            plsc.store_compressed(ci_i.at[pl.ds(ni, 16)], iv, mask=mi)
            plsc.store_compressed(ci_b.at[pl.ds(ni, 16)], bvec, mask=mi)
            ni = ni + plsc.all_reduce_population_count(mi)[0]
            return nu, ni

        return lax.fori_loop(0, 2048 // 16, cand, carry)

    nu, ni = lax.fori_loop(0, _B // 2048, cand_chunk,
                           (jnp.int32(0), jnp.int32(0)))
    sent = jnp.full((16,), _SENT, jnp.int32)
    zero16 = jnp.zeros((16,), jnp.int32)
    cu_i[pl.ds(nu, 16)] = sent
    cu_b[pl.ds(nu, 16)] = zero16
    ci_i[pl.ds(ni, 16)] = sent
    ci_b[pl.ds(ni, 16)] = zero16

    def part(c_i, c_b, nc, g_i, g_b):
        counts = []
        for g in range(_G):
            glo = wlo + (nw * g) // _G
            ghi = wlo + (nw * (g + 1)) // _G

            def one(k, ng):
                sl = pl.ds(k * 16, 16)
                cv = c_i[sl]
                cb = c_b[sl]
                wv = lax.shift_right_logical(cv, 10)
                m = (wv >= glo) & (wv < ghi)
                plsc.store_compressed(g_i.at[pl.ds(g * _CG + ng, 16)], cv,
                                      mask=m)
                plsc.store_compressed(g_b.at[pl.ds(g * _CG + ng, 16)], cb,
                                      mask=m)
                return ng + plsc.all_reduce_population_count(m)[0]

            ng = lax.fori_loop(0, (nc + 15) // 16, one, jnp.int32(0))
            g_i[pl.ds(g * _CG + ng, 16)] = sent
            g_b[pl.ds(g * _CG + ng, 16)] = zero16
            counts.append(ng)
        return counts

    ucnt = part(cu_i, cu_b, nu, gu_i, gu_b)
    icnt = part(ci_i, ci_b, ni, gi_i, gi_b)

    glos = [wlo + (nw * g) // _G for g in range(1, _G)]

    def group_of(w):
        g = jnp.int32(0)
        for b in glos:
            g = g + jnp.where(w >= b, 1, 0).astype(jnp.int32)
        return g

    def count_for(counts, g):
        c = counts[0]
        for gg in range(1, _G):
            c = jnp.where(g == gg, counts[gg], c)
        return c

    def fire_u(w, buf):
        off = pl.multiple_of(w * _W, 128)
        pltpu.async_copy(ut_hbm.at[:, pl.ds(off, _W)], win_v.at[buf], sem)

    def fire_i(w, buf):
        off = pl.multiple_of(w * _W, 128)
        pltpu.async_copy(it_hbm.at[:, pl.ds(off, _W)], win_v.at[buf], sem_i)

    def wait_slab(buf):
        s = sem if buf == 0 else sem_i
        pltpu.make_async_copy(ut_hbm.at[:, pl.ds(0, _W)], win_v.at[buf],
                              s).wait()

    def process(w, wbase, buf, g_i, g_b, counts, sh):
        g = group_of(w)
        g0 = g * _CG
        nc = count_for(counts, g)

        def scan_c(k, nh):
            sl = pl.ds(g0 + k * 16, 16)
            cv = g_i[sl]
            cb = g_b[sl]
            m = lax.shift_right_logical(cv, 10) == w
            plsc.store_compressed(hw_i.at[pl.ds(nh, 16)], cv, mask=m)
            plsc.store_compressed(hw_b.at[pl.ds(nh, 16)], cb, mask=m)
            return nh + plsc.all_reduce_population_count(m)[0]

        nh = lax.fori_loop(0, (nc + 15) // 16, scan_c, jnp.int32(0))
        hw_i[pl.ds(nh, 16)] = jnp.full((16,), wbase, jnp.int32)
        hw_b[pl.ds(nh, 16)] = jnp.full((16,), _B, jnp.int32)

        def hits(h, carry):
            hv = hw_i[pl.ds(h * 8, 16)]
            hb = hw_b[pl.ds(h * 8, 16)]
            col = hv - wbase
            for r in range(8):
                vec = plsc.load_gather(
                    win_v.at[buf],
                    [lane, jnp.full((16,), col[r], jnp.int32)])
                stage_v[pl.ds(r * 16, 16)] = vec
                sidx_v[pl.ds(r * 16, 16)] = hb[r] * 16 + lane
            pltpu.sync_copy(stage_v, sh.at[sidx_v])
            return carry

        lax.fori_loop(0, (nh + 7) // 8, hits, 0)

    @pl.when(wlo < whi_full)
    def _():
        fire_u(wlo, 0)
        fire_i(wlo, 1)

    def window(k, carry):
        w = wlo + k
        wait_slab(0)
        process(w, w * _W, 0, gu_i, gu_b, ucnt, shu)

        @pl.when(k + 1 < whi_full - wlo)
        def _():
            fire_u(w + 1, 0)

        wait_slab(1)
        process(w, w * _W, 1, gi_i, gi_b, icnt, shi)

        @pl.when(k + 1 < whi_full - wlo)
        def _():
            fire_i(w + 1, 1)

        return carry

    lax.fori_loop(0, whi_full - wlo, window, 0)

    @pl.when(whi == _NWIN)
    def _():
        pltpu.sync_copy(ut_hbm.at[:, pl.ds(_RAG0, _RAGW)],
                        win_v.at[0, :, pl.ds(0, _RAGW)])
        process(jnp.int32(_NFULL), _RAG0, 0, gu_i, gu_b, ucnt, shu)
        pltpu.sync_copy(it_hbm.at[:, pl.ds(_RAG0, _RAGW)],
                        win_v.at[1, :, pl.ds(0, _RAGW)])
        process(jnp.int32(_NFULL), _RAG0, 1, gi_i, gi_b, icnt, shi)

    plsc.subcore_barrier()
    seg = _IMG // 16
    for sh, out in ((shu, ue_hbm), (shi, ve_hbm)):
        for c in range(5):
            off = c * 4096
            sz = 4096 if c < 4 else 16
            bsl = pl.ds(0, sz)
            pltpu.sync_copy(sh.at[pl.ds(t * seg + off, sz)], bounce_v.at[bsl])
            pltpu.sync_copy(bounce_v.at[bsl],
                            out.at[pl.ds(cid * _IMG + t * seg + off, sz)])


def _dot_body(uidx_hbm, iidx_hbm, ue_hbm, ve_hbm, utail_hbm, itail_hbm,
              ubt_hbm, ibt_hbm, gb_hbm,
              out_hbm,
              uidx_v, iidx_v, ue0_l, ue1_l, ve0_l, ve1_l,
              ub_v, ib_v, out_v, gb_v, utail_v, itail_v, sem):
    wid = lax.axis_index("s") * 2 + lax.axis_index("c")
    base = wid * _BPW
    lane = lax.iota(jnp.int32, 16)

    pltpu.sync_copy(uidx_hbm.at[pl.ds(base, _BPW)], uidx_v)
    pltpu.sync_copy(iidx_hbm.at[pl.ds(base, _BPW)], iidx_v)
    esl0 = pl.ds(base * 16, _BPW * 16)
    esl1 = pl.ds(_IMG + base * 16, _BPW * 16)
    pltpu.sync_copy(ue_hbm.at[esl0], ue0_l)
    pltpu.sync_copy(ue_hbm.at[esl1], ue1_l)
    pltpu.sync_copy(ve_hbm.at[esl0], ve0_l)
    pltpu.sync_copy(ve_hbm.at[esl1], ve1_l)
    pltpu.sync_copy(gb_hbm, gb_v)
    pltpu.sync_copy(utail_hbm, utail_v)
    pltpu.sync_copy(itail_hbm, itail_v)
    copies = []
    for c in range(_BPW // 128):
        sl = pl.ds(c * 128, 128)
        copies.append(
            pltpu.async_copy(ubt_hbm.at[uidx_v.at[sl]], ub_v.at[sl], sem))
        copies.append(
            pltpu.async_copy(ibt_hbm.at[iidx_v.at[sl]], ib_v.at[sl], sem))
    for cp in copies:
        cp.wait()
    gb = gb_v[...]

    def block(j, carry):
        r0 = j * 16
        uiv = uidx_v[pl.ds(r0, 16)]
        iiv = iidx_v[pl.ds(r0, 16)]
        acc = gb
        for r in range(16):
            rsl = pl.ds((r0 + r) * 16, 16)
            u = jnp.where(uiv[r] < _SPLIT, ue0_l[rsl], ue1_l[rsl])
            v = jnp.where(iiv[r] < _SPLIT, ve0_l[rsl], ve1_l[rsl])
            ut_ix = jnp.maximum(uiv[r] - _TAIL0, 0) * 16 + lane
            it_ix = jnp.maximum(iiv[r] - _TAIL0, 0) * 16 + lane
            u = jnp.where(uiv[r] >= _TAIL0,
                          plsc.load_gather(utail_v, [ut_ix]), u)
            v = jnp.where(iiv[r] >= _TAIL0,
                          plsc.load_gather(itail_v, [it_ix]), v)
            s = jnp.sum(u * v)
            acc = jnp.where(lane == r, acc + s, acc)
        sl = pl.ds(r0, 16)
        out_v[sl] = acc + ub_v[sl] + ib_v[sl]
        return carry

    lax.fori_loop(0, _BPW // 16, block, 0)
    pltpu.sync_copy(out_v, out_hbm.at[pl.ds(base, _BPW)])


def kernel(user_idx, item_idx, user_table, item_table, implicit_table,
           user_bias_table, item_bias_table, global_bias):
    del implicit_table
    uidx = user_idx.astype(jnp.int32)
    iidx = item_idx.astype(jnp.int32)
    ut = user_table.T
    it = item_table.T
    utail = user_table[_TAIL0:].reshape(-1)
    itail = item_table[_TAIL0:].reshape(-1)
    ubt = user_bias_table.reshape(-1)
    ibt = item_bias_table.reshape(-1)
    gb16 = jnp.broadcast_to(global_bias, (16,)).astype(jnp.float32)

    mesh = plsc.VectorSubcoreMesh(core_axis_name="c", subcore_axis_name="s")
    params = pltpu.CompilerParams(needs_layout_passes=False)
    scan = pl.kernel(
        _scan_body,
        out_type=(jax.ShapeDtypeStruct((2 * _IMG,), jnp.float32),
                  jax.ShapeDtypeStruct((2 * _IMG,), jnp.float32)),
        mesh=mesh,
        compiler_params=params,
        scratch_types=[
            pltpu.VMEM((2048,), jnp.int32),
            pltpu.VMEM((2048,), jnp.int32),
            pltpu.VMEM((2, _F, _W), jnp.float32),
            pltpu.VMEM((1168,), jnp.int32),
            pltpu.VMEM((1168,), jnp.int32),
            pltpu.VMEM((1168,), jnp.int32),
            pltpu.VMEM((1168,), jnp.int32),
            pltpu.VMEM((_G * _CG + 16,), jnp.int32),
            pltpu.VMEM((_G * _CG + 16,), jnp.int32),
            pltpu.VMEM((_G * _CG + 16,), jnp.int32),
            pltpu.VMEM((_G * _CG + 16,), jnp.int32),
            pltpu.VMEM((528,), jnp.int32),
            pltpu.VMEM((528,), jnp.int32),
            pltpu.VMEM((128,), jnp.float32),
            pltpu.VMEM((128,), jnp.int32),
            pltpu.VMEM((4096,), jnp.float32),
            pltpu.VMEM_SHARED((_IMG,), jnp.float32),
            pltpu.VMEM_SHARED((_IMG,), jnp.float32),
            pltpu.SemaphoreType.DMA,
            pltpu.SemaphoreType.DMA,
        ],
    )
    ue, ve = scan(uidx, iidx, ut, it)
    dot = pl.kernel(
        _dot_body,
        out_type=jax.ShapeDtypeStruct((_B,), jnp.float32),
        mesh=mesh,
        compiler_params=params,
        scratch_types=[
            pltpu.VMEM((_BPW,), jnp.int32),
            pltpu.VMEM((_BPW,), jnp.int32),
            pltpu.VMEM((_BPW * _F,), jnp.float32),
            pltpu.VMEM((_BPW * _F,), jnp.float32),
            pltpu.VMEM((_BPW * _F,), jnp.float32),
            pltpu.VMEM((_BPW * _F,), jnp.float32),
            pltpu.VMEM((_BPW,), jnp.float32),
            pltpu.VMEM((_BPW,), jnp.float32),
            pltpu.VMEM((_BPW,), jnp.float32),
            pltpu.VMEM((16,), jnp.float32),
            pltpu.VMEM((64 * _F,), jnp.float32),
            pltpu.VMEM((64 * _F,), jnp.float32),
            pltpu.SemaphoreType.DMA,
        ],
    )
    return dot(uidx, iidx, ue, ve, utail, itail, ubt, ibt, gb16)


# speedup vs baseline: 68.0531x; 1.0831x over previous
import functools
import jax
import jax.numpy as jnp
from jax import lax
from jax.experimental import pallas as pl
from jax.experimental.pallas import tpu as pltpu
from jax.experimental.pallas import tpu_sc as plsc

_B = 16384
_F = 16
_NW = 32
_BPW = _B // _NW
_W = 1024
_NFULL = 976
_RAG0 = _NFULL * _W
_RAGW = 512
_TAIL0 = _RAG0 + _RAGW
_NWIN = _NFULL + 1
_HALF = 488
_SPLIT = _HALF * _W
_SENT = 1 << 30
_IMG = (_B + 16) * _F
_G = 8
_CG = 768


def _scan_body(uidx_hbm, iidx_hbm, ut_hbm, it_hbm,
               ue_hbm, ve_hbm,
               uidx_v, iidx_v, uidx2_v, iidx2_v, win_v, cu_i, cu_b, ci_i, ci_b,
               gu_i, gu_b, gi_i, gi_b,
               hw_i, hw_b, stage_v, sidx_v, bounce_v,
               shu, shi,
               sem, sem_i):
    t = lax.axis_index("s")
    cid = lax.axis_index("c")
    nwin_sc = jnp.where(cid == 0, _HALF, _NWIN - _HALF)
    sc0 = cid * _HALF
    wlo = sc0 + (nwin_sc * t) // 16
    whi = sc0 + (nwin_sc * (t + 1)) // 16
    whi_full = jnp.minimum(whi, _NFULL)
    nw = whi - wlo
    lane = lax.iota(jnp.int32, 16)

    def fire_idx(ch, ubuf, ibuf):
        sl = pl.ds(ch * 2048, 2048)
        pltpu.async_copy(uidx_hbm.at[sl], ubuf, sem)
        pltpu.async_copy(iidx_hbm.at[sl], ibuf, sem_i)

    def wait_idx(ubuf, ibuf):
        pltpu.make_async_copy(uidx_hbm.at[pl.ds(0, 2048)], ubuf, sem).wait()
        pltpu.make_async_copy(iidx_hbm.at[pl.ds(0, 2048)], ibuf, sem_i).wait()

    def cand_for(ch, ubuf, ibuf, carry):
        def cand(k, carry2):
            nu, ni = carry2
            sl = pl.ds(k * 16, 16)
            bvec = lane + (ch * 2048 + k * 16)
            uv = ubuf[sl]
            wv = lax.shift_right_logical(uv, 10)
            mu = (wv >= wlo) & (wv < whi) & (uv < _TAIL0)
            plsc.store_compressed(cu_i.at[pl.ds(nu, 16)], uv, mask=mu)
            plsc.store_compressed(cu_b.at[pl.ds(nu, 16)], bvec, mask=mu)
            nu = nu + plsc.all_reduce_population_count(mu)[0]
            iv = ibuf[sl]
            wv = lax.shift_right_logical(iv, 10)
            mi = (wv >= wlo) & (wv < whi) & (iv < _TAIL0)
            plsc.store_compressed(ci_i.at[pl.ds(ni, 16)], iv, mask=mi)
            plsc.store_compressed(ci_b.at[pl.ds(ni, 16)], bvec, mask=mi)
            ni = ni + plsc.all_reduce_population_count(mi)[0]
            return nu, ni

        return lax.fori_loop(0, 2048 // 16, cand, carry)

    fire_idx(0, uidx_v, iidx_v)

    def cand_pair(p, carry):
        ch = p * 2
        wait_idx(uidx_v, iidx_v)
        fire_idx(ch + 1, uidx2_v, iidx2_v)
        carry = cand_for(ch, uidx_v, iidx_v, carry)
        wait_idx(uidx2_v, iidx2_v)

        @pl.when(ch + 2 < _B // 2048)
        def _():
            fire_idx(ch + 2, uidx_v, iidx_v)

        return cand_for(ch + 1, uidx2_v, iidx2_v, carry)

    nu, ni = lax.fori_loop(0, _B // 4096, cand_pair,
                           (jnp.int32(0), jnp.int32(0)))
    sent = jnp.full((16,), _SENT, jnp.int32)
    zero16 = jnp.zeros((16,), jnp.int32)
    cu_i[pl.ds(nu, 16)] = sent
    cu_b[pl.ds(nu, 16)] = zero16
    ci_i[pl.ds(ni, 16)] = sent
    ci_b[pl.ds(ni, 16)] = zero16

    def part(c_i, c_b, nc, g_i, g_b):
        counts = []
        for g in range(_G):
            glo = wlo + (nw * g) // _G
            ghi = wlo + (nw * (g + 1)) // _G

            def one(k, ng):
                sl = pl.ds(k * 16, 16)
                cv = c_i[sl]
                cb = c_b[sl]
                wv = lax.shift_right_logical(cv, 10)
                m = (wv >= glo) & (wv < ghi)
                plsc.store_compressed(g_i.at[pl.ds(g * _CG + ng, 16)], cv,
                                      mask=m)
                plsc.store_compressed(g_b.at[pl.ds(g * _CG + ng, 16)], cb,
                                      mask=m)
                return ng + plsc.all_reduce_population_count(m)[0]

            ng = lax.fori_loop(0, (nc + 15) // 16, one, jnp.int32(0))
            g_i[pl.ds(g * _CG + ng, 16)] = sent
            g_b[pl.ds(g * _CG + ng, 16)] = zero16
            counts.append(ng)
        return counts

    ucnt = part(cu_i, cu_b, nu, gu_i, gu_b)
    icnt = part(ci_i, ci_b, ni, gi_i, gi_b)

    glos = [wlo + (nw * g) // _G for g in range(1, _G)]

    def group_of(w):
        g = jnp.int32(0)
        for b in glos:
            g = g + jnp.where(w >= b, 1, 0).astype(jnp.int32)
        return g

    def count_for(counts, g):
        c = counts[0]
        for gg in range(1, _G):
            c = jnp.where(g == gg, counts[gg], c)
        return c

    def fire_u(w, buf):
        off = pl.multiple_of(w * _W, 128)
        pltpu.async_copy(ut_hbm.at[:, pl.ds(off, _W)], win_v.at[buf], sem)

    def fire_i(w, buf):
        off = pl.multiple_of(w * _W, 128)
        pltpu.async_copy(it_hbm.at[:, pl.ds(off, _W)], win_v.at[buf], sem_i)

    def wait_slab(buf):
        s = sem if buf == 0 else sem_i
        pltpu.make_async_copy(ut_hbm.at[:, pl.ds(0, _W)], win_v.at[buf],
                              s).wait()

    def process(w, wbase, buf, g_i, g_b, counts, sh):
        g = group_of(w)
        g0 = g * _CG
        nc = count_for(counts, g)

        def scan_c(k, nh):
            sl = pl.ds(g0 + k * 16, 16)
            cv = g_i[sl]
            cb = g_b[sl]
            m = lax.shift_right_logical(cv, 10) == w
            plsc.store_compressed(hw_i.at[pl.ds(nh, 16)], cv, mask=m)
            plsc.store_compressed(hw_b.at[pl.ds(nh, 16)], cb, mask=m)
            return nh + plsc.all_reduce_population_count(m)[0]

        nh = lax.fori_loop(0, (nc + 15) // 16, scan_c, jnp.int32(0))
        hw_i[pl.ds(nh, 16)] = jnp.full((16,), wbase, jnp.int32)
        hw_b[pl.ds(nh, 16)] = jnp.full((16,), _B, jnp.int32)

        def hits(h, carry):
            hv = hw_i[pl.ds(h * 8, 16)]
            hb = hw_b[pl.ds(h * 8, 16)]
            col = hv - wbase
            for r in range(8):
                vec = plsc.load_gather(
                    win_v.at[buf],
                    [lane, jnp.full((16,), col[r], jnp.int32)])
                stage_v[pl.ds(r * 16, 16)] = vec
                sidx_v[pl.ds(r * 16, 16)] = hb[r] * 16 + lane
            pltpu.sync_copy(stage_v, sh.at[sidx_v])
            return carry

        lax.fori_loop(0, (nh + 7) // 8, hits, 0)

    @pl.when(wlo < whi_full)
    def _():
        fire_u(wlo, 0)
        fire_i(wlo, 1)

    def window(k, carry):
        w = wlo + k
        wait_slab(0)
        process(w, w * _W, 0, gu_i, gu_b, ucnt, shu)

        @pl.when(k + 1 < whi_full - wlo)
        def _():
            fire_u(w + 1, 0)

        wait_slab(1)
        process(w, w * _W, 1, gi_i, gi_b, icnt, shi)

        @pl.when(k + 1 < whi_full - wlo)
        def _():
            fire_i(w + 1, 1)

        return carry

    lax.fori_loop(0, whi_full - wlo, window, 0)

    @pl.when(whi == _NWIN)
    def _():
        pltpu.sync_copy(ut_hbm.at[:, pl.ds(_RAG0, _RAGW)],
                        win_v.at[0, :, pl.ds(0, _RAGW)])
        process(jnp.int32(_NFULL), _RAG0, 0, gu_i, gu_b, ucnt, shu)
        pltpu.sync_copy(it_hbm.at[:, pl.ds(_RAG0, _RAGW)],
                        win_v.at[1, :, pl.ds(0, _RAGW)])
        process(jnp.int32(_NFULL), _RAG0, 1, gi_i, gi_b, icnt, shi)

    plsc.subcore_barrier()
    seg = _IMG // 16
    for sh, out in ((shu, ue_hbm), (shi, ve_hbm)):
        for c in range(5):
            off = c * 4096
            sz = 4096 if c < 4 else 16
            bsl = pl.ds(0, sz)
            pltpu.sync_copy(sh.at[pl.ds(t * seg + off, sz)], bounce_v.at[bsl])
            pltpu.sync_copy(bounce_v.at[bsl],
                            out.at[pl.ds(cid * _IMG + t * seg + off, sz)])


def _dot_body(uidx_hbm, iidx_hbm, ue_hbm, ve_hbm, utail_hbm, itail_hbm,
              ubt_hbm, ibt_hbm, gb_hbm,
              out_hbm,
              uidx_v, iidx_v, ue0_l, ue1_l, ve0_l, ve1_l,
              ub_v, ib_v, out_v, gb_v, utail_v, itail_v, sem):
    wid = lax.axis_index("s") * 2 + lax.axis_index("c")
    base = wid * _BPW
    lane = lax.iota(jnp.int32, 16)

    pltpu.sync_copy(uidx_hbm.at[pl.ds(base, _BPW)], uidx_v)
    pltpu.sync_copy(iidx_hbm.at[pl.ds(base, _BPW)], iidx_v)
    esl0 = pl.ds(base * 16, _BPW * 16)
    esl1 = pl.ds(_IMG + base * 16, _BPW * 16)
    pltpu.sync_copy(ue_hbm.at[esl0], ue0_l)
    pltpu.sync_copy(ue_hbm.at[esl1], ue1_l)
    pltpu.sync_copy(ve_hbm.at[esl0], ve0_l)
    pltpu.sync_copy(ve_hbm.at[esl1], ve1_l)
    pltpu.sync_copy(gb_hbm, gb_v)
    pltpu.sync_copy(utail_hbm, utail_v)
    pltpu.sync_copy(itail_hbm, itail_v)
    copies = []
    for c in range(_BPW // 128):
        sl = pl.ds(c * 128, 128)
        copies.append(
            pltpu.async_copy(ubt_hbm.at[uidx_v.at[sl]], ub_v.at[sl], sem))
        copies.append(
            pltpu.async_copy(ibt_hbm.at[iidx_v.at[sl]], ib_v.at[sl], sem))
    for cp in copies:
        cp.wait()
    gb = gb_v[...]

    def block(j, carry):
        r0 = j * 16
        uiv = uidx_v[pl.ds(r0, 16)]
        iiv = iidx_v[pl.ds(r0, 16)]
        acc = gb
        for r in range(16):
            rsl = pl.ds((r0 + r) * 16, 16)
            u = jnp.where(uiv[r] < _SPLIT, ue0_l[rsl], ue1_l[rsl])
            v = jnp.where(iiv[r] < _SPLIT, ve0_l[rsl], ve1_l[rsl])
            ut_ix = jnp.maximum(uiv[r] - _TAIL0, 0) * 16 + lane
            it_ix = jnp.maximum(iiv[r] - _TAIL0, 0) * 16 + lane
            u = jnp.where(uiv[r] >= _TAIL0,
                          plsc.load_gather(utail_v, [ut_ix]), u)
            v = jnp.where(iiv[r] >= _TAIL0,
                          plsc.load_gather(itail_v, [it_ix]), v)
            s = jnp.sum(u * v)
            acc = jnp.where(lane == r, acc + s, acc)
        sl = pl.ds(r0, 16)
        out_v[sl] = acc + ub_v[sl] + ib_v[sl]
        return carry

    lax.fori_loop(0, _BPW // 16, block, 0)
    pltpu.sync_copy(out_v, out_hbm.at[pl.ds(base, _BPW)])


def kernel(user_idx, item_idx, user_table, item_table, implicit_table,
           user_bias_table, item_bias_table, global_bias):
    del implicit_table
    uidx = user_idx.astype(jnp.int32)
    iidx = item_idx.astype(jnp.int32)
    ut = user_table.T
    it = item_table.T
    utail = user_table[_TAIL0:].reshape(-1)
    itail = item_table[_TAIL0:].reshape(-1)
    ubt = user_bias_table.reshape(-1)
    ibt = item_bias_table.reshape(-1)
    gb16 = jnp.broadcast_to(global_bias, (16,)).astype(jnp.float32)

    mesh = plsc.VectorSubcoreMesh(core_axis_name="c", subcore_axis_name="s")
    params = pltpu.CompilerParams(needs_layout_passes=False)
    scan = pl.kernel(
        _scan_body,
        out_type=(jax.ShapeDtypeStruct((2 * _IMG,), jnp.float32),
                  jax.ShapeDtypeStruct((2 * _IMG,), jnp.float32)),
        mesh=mesh,
        compiler_params=params,
        scratch_types=[
            pltpu.VMEM((2048,), jnp.int32),
            pltpu.VMEM((2048,), jnp.int32),
            pltpu.VMEM((2048,), jnp.int32),
            pltpu.VMEM((2048,), jnp.int32),
            pltpu.VMEM((2, _F, _W), jnp.float32),
            pltpu.VMEM((1168,), jnp.int32),
            pltpu.VMEM((1168,), jnp.int32),
            pltpu.VMEM((1168,), jnp.int32),
            pltpu.VMEM((1168,), jnp.int32),
            pltpu.VMEM((_G * _CG + 16,), jnp.int32),
            pltpu.VMEM((_G * _CG + 16,), jnp.int32),
            pltpu.VMEM((_G * _CG + 16,), jnp.int32),
            pltpu.VMEM((_G * _CG + 16,), jnp.int32),
            pltpu.VMEM((528,), jnp.int32),
            pltpu.VMEM((528,), jnp.int32),
            pltpu.VMEM((128,), jnp.float32),
            pltpu.VMEM((128,), jnp.int32),
            pltpu.VMEM((4096,), jnp.float32),
            pltpu.VMEM_SHARED((_IMG,), jnp.float32),
            pltpu.VMEM_SHARED((_IMG,), jnp.float32),
            pltpu.SemaphoreType.DMA,
            pltpu.SemaphoreType.DMA,
        ],
    )
    ue, ve = scan(uidx, iidx, ut, it)
    dot = pl.kernel(
        _dot_body,
        out_type=jax.ShapeDtypeStruct((_B,), jnp.float32),
        mesh=mesh,
        compiler_params=params,
        scratch_types=[
            pltpu.VMEM((_BPW,), jnp.int32),
            pltpu.VMEM((_BPW,), jnp.int32),
            pltpu.VMEM((_BPW * _F,), jnp.float32),
            pltpu.VMEM((_BPW * _F,), jnp.float32),
            pltpu.VMEM((_BPW * _F,), jnp.float32),
            pltpu.VMEM((_BPW * _F,), jnp.float32),
            pltpu.VMEM((_BPW,), jnp.float32),
            pltpu.VMEM((_BPW,), jnp.float32),
            pltpu.VMEM((_BPW,), jnp.float32),
            pltpu.VMEM((16,), jnp.float32),
            pltpu.VMEM((64 * _F,), jnp.float32),
            pltpu.VMEM((64 * _F,), jnp.float32),
            pltpu.SemaphoreType.DMA,
        ],
    )
    return dot(uidx, iidx, ue, ve, utail, itail, ubt, ibt, gb16)
